# baseline (speedup 1.0000x reference)
import functools

import jax
import jax.numpy as jnp
from jax import lax
from jax.experimental import pallas as pl
from jax.experimental.pallas import tpu as pltpu
from jax.experimental.pallas import tpu_sc as plsc

VOCAB = 100000
SEQ = 200
DIM = 64
BATCH = 4096

_NC = 2
_NS = 16
_NW = _NC * _NS
_BPW = BATCH // _NW
_H = SEQ // 2
_HP = 56


def _gather_body(x_hbm, tbl_hbm, pos_hbm, out_hbm,
                 idx_v, pos_v, gb0, gb1, ob0, ob1, g0, g1, o0, o1):
    wid = lax.axis_index("s") * _NC + lax.axis_index("c")
    b0 = wid * _BPW

    gbuf = (gb0, gb1)
    obuf = (ob0, ob1)
    gsem = (g0, g1)
    osem = (o0, o1)

    pltpu.sync_copy(pos_hbm, pos_v)
    pltpu.sync_copy(x_hbm.at[pl.ds(b0, _BPW)], idx_v)

    def gather(i, par):
        return [pltpu.make_async_copy(
            tbl_hbm.at[idx_v.at[i, k]], gbuf[par].at[k], gsem[par])
            for k in range(2)]

    def outcopy(i, par):
        return pltpu.make_async_copy(
            obuf[par], out_hbm.at[b0 + i, :, pl.ds(0, _H // 2)], osem[par])

    for cp in gather(0, 0):
        cp.start()

    @pl.loop(0, _BPW, step=2)
    def _row(g):
        for par in range(2):
            i = g + par
            nxt = 1 - par

            @pl.when(i + 1 < _BPW)
            def _():
                for cp in gather(i + 1, nxt):
                    cp.start()

            for cp in gather(i, par):
                cp.wait()

            @pl.when(i >= 2)
            def _():
                outcopy(i - 2, par).wait()

            @pl.loop(0, _H // 2)
            def _pos(u):
                for k in range(2):
                    for h in range(2):
                        for j in range(DIM // 16):
                            sl = pl.ds(j * 16, 16)
                            obuf[par][k, u, pl.ds(h * DIM + j * 16, 16)] = (
                                gbuf[par][k, 2 * u + h, sl]
                                + pos_v[k, 2 * u + h, sl])

            outcopy(i, par).start()

    outcopy(_BPW - 2, 0).wait()
    outcopy(_BPW - 1, 1).wait()


def _transpose_body(x_ref, o_ref):
    eye = (lax.broadcasted_iota(jnp.int32, (128, 128), 0)
           == lax.broadcasted_iota(jnp.int32, (128, 128), 1)
           ).astype(jnp.float32)

    def body(u, carry):
        for k in range(2):
            xm = x_ref[:, k, u, :]
            y = lax.dot_general(xm, eye, (((0,), (0,)), ((), ())),
                                preferred_element_type=jnp.float32)
            o_ref[pl.ds(k * SEQ // 2 + 2 * u, 2), :, 0, :, :] = (
                y.reshape(2, 8, 8, 128))
        return carry

    lax.fori_loop(0, _H // 2, body, 0)


@jax.jit
def _tpe(x3, token_table, pos3):
    sc = functools.partial(
        pl.kernel,
        out_type=jax.ShapeDtypeStruct((BATCH, 2, _HP, 128), jnp.float32),
        mesh=plsc.VectorSubcoreMesh(core_axis_name="c", subcore_axis_name="s"),
        scratch_types=[
            pltpu.VMEM((_BPW, 2, _H), jnp.int32),
            pltpu.VMEM((2, _H, DIM), jnp.float32),
            pltpu.VMEM((2, _H, DIM), jnp.float32),
            pltpu.VMEM((2, _H, DIM), jnp.float32),
            pltpu.VMEM((2, _H // 2, 128), jnp.float32),
            pltpu.VMEM((2, _H // 2, 128), jnp.float32),
            pltpu.SemaphoreType.DMA,
            pltpu.SemaphoreType.DMA,
            pltpu.SemaphoreType.DMA,
            pltpu.SemaphoreType.DMA,
        ],
        compiler_params=pltpu.CompilerParams(use_tc_tiling_on_sc=False),
    )(_gather_body)
    lin = sc(x3, token_table, pos3)

    tc = pl.pallas_call(
        _transpose_body,
        out_shape=jax.ShapeDtypeStruct((SEQ, 8, _NW, 8, 128), jnp.float32),
        grid=(_NW,),
        in_specs=[pl.BlockSpec((_BPW, 2, _HP, 128), lambda w: (w, 0, 0, 0))],
        out_specs=pl.BlockSpec((SEQ, 8, 1, 8, 128), lambda w: (0, 0, w, 0, 0)),
        compiler_params=pltpu.CompilerParams(
            dimension_semantics=("arbitrary",)),
    )
    return tc(lin)


def kernel(x, token_table, pos_table):
    x3 = x.reshape(BATCH, 2, _H).astype(jnp.int32)
    pos3 = pos_table.reshape(2, _H, DIM)
    out = _tpe(x3, token_table, pos3)
    return out.transpose(2, 4, 0, 1, 3).reshape(BATCH, SEQ, DIM)


# speedup vs baseline: 1.2683x; 1.2683x over previous
import functools

import jax
import jax.numpy as jnp
from jax import lax
from jax.experimental import pallas as pl
from jax.experimental.pallas import tpu as pltpu
from jax.experimental.pallas import tpu_sc as plsc

VOCAB = 100000
SEQ = 200
DIM = 64
BATCH = 4096

_NC = 2
_NS = 16
_NW = _NC * _NS
_BPW = BATCH // _NW
_H = SEQ // 2
_HP = 56


def _gather_body(x_hbm, tbl_hbm, pos_hbm, out_hbm,
                 idx_v, pos_v, gb0, gb1, ob0, ob1, g0, g1, o0, o1):
    wid = lax.axis_index("s") * _NC + lax.axis_index("c")
    b0 = wid * _BPW

    gbuf = (gb0, gb1)
    obuf = (ob0, ob1)
    gsem = (g0, g1)
    osem = (o0, o1)

    pltpu.sync_copy(pos_hbm, pos_v)
    pltpu.sync_copy(x_hbm.at[pl.ds(b0, _BPW)], idx_v)

    def gather(i, par):
        return pltpu.make_async_copy(
            tbl_hbm.at[idx_v.at[i]], gbuf[par], gsem[par])

    def outcopy(i, par):
        return pltpu.make_async_copy(
            obuf[par], out_hbm.at[b0 + i, pl.ds(0, _H // 2)], osem[par])

    gather(0, 0).start()

    @pl.loop(0, _BPW, step=2)
    def _row(g):
        for par in range(2):
            i = g + par
            nxt = 1 - par

            @pl.when(i + 1 < _BPW)
            def _():
                gather(i + 1, nxt).start()

            gather(i, par).wait()

            @pl.when(i >= 2)
            def _():
                outcopy(i - 2, par).wait()

            @pl.loop(0, _H // 2)
            def _pos(u):
                for h in range(2):
                    for j in range(DIM // 16):
                        sl = pl.ds(j * 16, 16)
                        obuf[par][u, pl.ds(h * DIM + j * 16, 16)] = (
                            gbuf[par][2 * u + h, sl] + pos_v[2 * u + h, sl])

            outcopy(i, par).start()

    outcopy(_BPW - 2, 0).wait()
    outcopy(_BPW - 1, 1).wait()


def _transpose_body(x_ref, o_ref):
    def body(u, carry):
        xm = x_ref[:, u, :]
        o_ref[pl.ds(2 * u, 2), :, 0, :, :] = xm.T.reshape(2, 8, 8, 128)
        return carry

    lax.fori_loop(0, _H // 2, body, 0)


def _transpose_alias_body(x_ref, prev_ref, o_ref):
    del prev_ref
    _transpose_body(x_ref, o_ref)


@jax.jit
def _tpe(x3, token_table, pos3):
    sc = functools.partial(
        pl.kernel,
        out_type=jax.ShapeDtypeStruct((BATCH, _HP, 128), jnp.float32),
        mesh=plsc.VectorSubcoreMesh(core_axis_name="c", subcore_axis_name="s"),
        scratch_types=[
            pltpu.VMEM((_BPW, _H), jnp.int32),
            pltpu.VMEM((_H, DIM), jnp.float32),
            pltpu.VMEM((_H, DIM), jnp.float32),
            pltpu.VMEM((_H, DIM), jnp.float32),
            pltpu.VMEM((_H // 2, 128), jnp.float32),
            pltpu.VMEM((_H // 2, 128), jnp.float32),
            pltpu.SemaphoreType.DMA,
            pltpu.SemaphoreType.DMA,
            pltpu.SemaphoreType.DMA,
            pltpu.SemaphoreType.DMA,
        ],
        compiler_params=pltpu.CompilerParams(use_tc_tiling_on_sc=False),
    )(_gather_body)

    lin0 = sc(x3[:, 0], token_table, pos3[0])
    lin1 = sc(x3[:, 1], token_table, pos3[1])

    out_sds = jax.ShapeDtypeStruct((SEQ, 8, _NW, 8, 128), jnp.float32)
    half_block = pl.BlockSpec((_H, 8, 1, 8, 128), lambda w: (0, 0, w, 0, 0))
    in_block = pl.BlockSpec((_BPW, _HP, 128), lambda w: (w, 0, 0))
    params = pltpu.CompilerParams(dimension_semantics=("arbitrary",))

    part = pl.pallas_call(
        _transpose_body,
        out_shape=out_sds,
        grid=(_NW,),
        in_specs=[in_block],
        out_specs=half_block,
        compiler_params=params,
    )(lin0)

    full = pl.pallas_call(
        _transpose_alias_body,
        out_shape=out_sds,
        grid=(_NW,),
        in_specs=[in_block, pl.BlockSpec(memory_space=pl.ANY)],
        out_specs=pl.BlockSpec((_H, 8, 1, 8, 128), lambda w: (1, 0, w, 0, 0)),
        input_output_aliases={1: 0},
        compiler_params=params,
    )(lin1, part)

    return full


def kernel(x, token_table, pos_table):
    x3 = x.reshape(BATCH, 2, _H).astype(jnp.int32)
    pos3 = pos_table.reshape(2, _H, DIM)
    out = _tpe(x3, token_table, pos3)
    return out.transpose(2, 4, 0, 1, 3).reshape(BATCH, SEQ, DIM)


# speedup vs baseline: 1.5714x; 1.2390x over previous
import functools

import jax
import jax.numpy as jnp
from jax import lax
from jax.experimental import pallas as pl
from jax.experimental.pallas import tpu as pltpu
from jax.experimental.pallas import tpu_sc as plsc

VOCAB = 100000
SEQ = 200
DIM = 64
BATCH = 4096

_NC = 2
_NS = 16
_NW = _NC * _NS
_NSLICE = 2
_SB = BATCH // _NSLICE
_BPW = _SB // _NW
_SW = _SB // 128
_H = SEQ // 2
_HP = 56


def _gather_body(x_hbm, tbl_hbm, pos_hbm, out_hbm,
                 idx_v, pos_v, gb0, gb1, ob0, ob1, g0, g1, o0, o1):
    wid = lax.axis_index("s") * _NC + lax.axis_index("c")
    b0 = wid * _BPW

    gbuf = (gb0, gb1)
    obuf = (ob0, ob1)
    gsem = (g0, g1)
    osem = (o0, o1)

    pltpu.sync_copy(pos_hbm, pos_v)
    pltpu.sync_copy(x_hbm.at[pl.ds(b0, _BPW)], idx_v)

    def gather(i, par):
        return [pltpu.make_async_copy(
            tbl_hbm.at[idx_v.at[i, k]], gbuf[par].at[k], gsem[par])
            for k in range(2)]

    def outcopy(i, par):
        return pltpu.make_async_copy(
            obuf[par], out_hbm.at[b0 + i, :, pl.ds(0, _H // 2)], osem[par])

    for cp in gather(0, 0):
        cp.start()

    @pl.loop(0, _BPW, step=2)
    def _row(g):
        for par in range(2):
            i = g + par
            nxt = 1 - par

            @pl.when(i + 1 < _BPW)
            def _():
                for cp in gather(i + 1, nxt):
                    cp.start()

            for cp in gather(i, par):
                cp.wait()

            @pl.when(i >= 2)
            def _():
                outcopy(i - 2, par).wait()

            @pl.loop(0, _H // 2)
            def _pos(u):
                for k in range(2):
                    for h in range(2):
                        for j in range(DIM // 16):
                            sl = pl.ds(j * 16, 16)
                            obuf[par][k, u, pl.ds(h * DIM + j * 16, 16)] = (
                                gbuf[par][k, 2 * u + h, sl]
                                + pos_v[k, 2 * u + h, sl])

            outcopy(i, par).start()

    outcopy(_BPW - 2, 0).wait()
    outcopy(_BPW - 1, 1).wait()


def _transpose_body(x_ref, o_ref):
    def body(u, carry):
        for k in range(2):
            xm = x_ref[:, k, u, :]
            o_ref[pl.ds(k * SEQ // 2 + 2 * u, 2), :, 0, :, :] = (
                xm.T.reshape(2, 8, 8, 128))
        return carry

    lax.fori_loop(0, _H // 2, body, 0)


def _transpose_alias_body(x_ref, prev_ref, o_ref):
    del prev_ref
    _transpose_body(x_ref, o_ref)


@jax.jit
def _tpe(x3, token_table, pos3):
    sc = functools.partial(
        pl.kernel,
        out_type=jax.ShapeDtypeStruct((_SB, 2, _HP, 128), jnp.float32),
        mesh=plsc.VectorSubcoreMesh(core_axis_name="c", subcore_axis_name="s"),
        scratch_types=[
            pltpu.VMEM((_BPW, 2, _H), jnp.int32),
            pltpu.VMEM((2, _H, DIM), jnp.float32),
            pltpu.VMEM((2, _H, DIM), jnp.float32),
            pltpu.VMEM((2, _H, DIM), jnp.float32),
            pltpu.VMEM((2, _H // 2, 128), jnp.float32),
            pltpu.VMEM((2, _H // 2, 128), jnp.float32),
            pltpu.SemaphoreType.DMA,
            pltpu.SemaphoreType.DMA,
            pltpu.SemaphoreType.DMA,
            pltpu.SemaphoreType.DMA,
        ],
        compiler_params=pltpu.CompilerParams(use_tc_tiling_on_sc=False),
    )(_gather_body)

    lins = [sc(x3[i * _SB:(i + 1) * _SB], token_table, pos3)
            for i in range(_NSLICE)]

    out_sds = jax.ShapeDtypeStruct((SEQ, 8, _NW, 8, 128), jnp.float32)
    in_block = pl.BlockSpec((128, 2, _HP, 128), lambda w: (w, 0, 0, 0))
    params = pltpu.CompilerParams(dimension_semantics=("arbitrary",))

    part = None
    for i in range(_NSLICE):
        out_block = pl.BlockSpec(
            (SEQ, 8, 1, 8, 128),
            functools.partial(lambda off, w: (0, 0, off + w, 0, 0), i * _SW))
        if part is None:
            part = pl.pallas_call(
                _transpose_body,
                out_shape=out_sds,
                grid=(_SW,),
                in_specs=[in_block],
                out_specs=out_block,
                compiler_params=params,
            )(lins[i])
        else:
            part = pl.pallas_call(
                _transpose_alias_body,
                out_shape=out_sds,
                grid=(_SW,),
                in_specs=[in_block, pl.BlockSpec(memory_space=pl.ANY)],
                out_specs=out_block,
                input_output_aliases={1: 0},
                compiler_params=params,
            )(lins[i], part)

    return part


def kernel(x, token_table, pos_table):
    x3 = x.reshape(BATCH, 2, _H).astype(jnp.int32)
    pos3 = pos_table.reshape(2, _H, DIM)
    out = _tpe(x3, token_table, pos3)
    return out.transpose(2, 4, 0, 1, 3).reshape(BATCH, SEQ, DIM)
